# baseline (speedup 1.0000x reference)
import functools

import numpy as np
import jax
import jax.numpy as jnp
from jax import lax
from jax.experimental import pallas as pl
from jax.experimental.pallas import tpu as pltpu
from jax.experimental.pallas import tpu_sc as plsc


def _pcall(*args, **kwargs):
    return pl.pallas_call(*args, **kwargs)


def _gather_rows(table, gidx):
    R, D = table.shape
    (Bt,) = gidx.shape
    info = plsc.get_sparse_core_info()
    NW = info.num_cores * info.num_subcores
    assert Bt % (8 * NW) == 0 and D % 16 == 0
    b_per_w = Bt // NW
    cap = max(8, min(128, (30000 // D) // 8 * 8))
    rc = min(b_per_w, cap)
    while b_per_w % rc:
        rc -= 8
    nloops = b_per_w // rc
    mesh = plsc.VectorSubcoreMesh(core_axis_name="c", subcore_axis_name="s")

    @functools.partial(
        pl.kernel, mesh=mesh,
        out_type=jax.ShapeDtypeStruct((Bt, D), jnp.float32),
        scratch_types=[
            pltpu.VMEM((rc,), jnp.int32),
            pltpu.VMEM((rc, D), jnp.float32),
            pltpu.SemaphoreType.DMA,
        ],
    )
    def k(table_hbm, idx_hbm, out_hbm, idx_v, rows_v, sem):
        wid = lax.axis_index("s") * info.num_cores + lax.axis_index("c")
        base = wid * b_per_w

        def one(i, carry):
            start = base + i * rc
            pltpu.sync_copy(idx_hbm.at[pl.ds(start, rc)], idx_v)
            pltpu.async_copy(table_hbm.at[idx_v], rows_v, sem).wait()
            pltpu.sync_copy(rows_v, out_hbm.at[pl.ds(start, rc)])
            return carry

        lax.fori_loop(0, nloops, one, 0)

    return k(table, gidx)


def _fps(xyz, npoint, fold):
    B, n, _ = xyz.shape
    nc = n // 8
    x3 = jnp.transpose(xyz, (0, 2, 1)).reshape(B, 3, 8, nc)
    fold_arr = jnp.full((8, 128), fold, jnp.int32)

    def body(fold_ref, x_ref, o_ref):
        x = x_ref[0, 0]
        y = x_ref[0, 1]
        z = x_ref[0, 2]
        gidx = (lax.broadcasted_iota(jnp.int32, (8, nc), 0) * nc
                + lax.broadcasted_iota(jnp.int32, (8, nc), 1))
        row_iota = lax.broadcasted_iota(jnp.int32, (1, npoint), 1)

        def step(i, carry):
            dists, far, row = carry
            row = jnp.where(row_iota == i, jnp.broadcast_to(far, row.shape), row)
            sel = gidx == far
            cx = jnp.sum(jnp.where(sel, x, 0.0))
            cy = jnp.sum(jnp.where(sel, y, 0.0))
            cz = jnp.sum(jnp.where(sel, z, 0.0))
            d = (x - cx) ** 2 + (y - cy) ** 2 + (z - cz) ** 2
            dists = jnp.minimum(dists, d)
            m = jnp.max(dists)
            far2 = jnp.min(jnp.where(dists == m, gidx, n)).reshape(1, 1)
            return dists, far2, row

        dists0 = jnp.full((8, nc), 1e10, jnp.float32)
        far0 = fold_ref[0:1, 0:1]
        row0 = jnp.zeros((1, npoint), jnp.int32)
        _, _, row = lax.fori_loop(0, npoint, step, (dists0, far0, row0))
        o_ref[0] = row + pl.program_id(0) * n

    out = _pcall(
        body,
        grid=(B,),
        in_specs=[
            pl.BlockSpec((8, 128), lambda b: (0, 0)),
            pl.BlockSpec((1, 3, 8, nc), lambda b: (b, 0, 0, 0)),
        ],
        out_specs=pl.BlockSpec((1, 1, npoint), lambda b: (b, 0, 0)),
        out_shape=jax.ShapeDtypeStruct((B, 1, npoint), jnp.int32),
    )(fold_arr, x3)
    return out.reshape(B, npoint)


def _ball_query(xyz, new_xyz, radius2, nsample):
    B, n, _ = xyz.shape
    P = new_xyz.shape[1]
    PC = min(128, P)
    xT = jnp.zeros((B, 8, n), jnp.float32).at[:, :3, :].set(
        jnp.transpose(xyz, (0, 2, 1)))
    nx8 = jnp.zeros((B, P, 8), jnp.float32).at[:, :, :3].set(new_xyz)

    def body(x_ref, nx_ref, o_ref):
        x = x_ref[0, 0:1, :]
        y = x_ref[0, 1:2, :]
        z = x_ref[0, 2:3, :]
        nx = nx_ref[0]
        cx = nx[:, 0:1]
        cy = nx[:, 1:2]
        cz = nx[:, 2:3]
        sqd = (cx - x) ** 2 + (cy - y) ** 2 + (cz - z) ** 2
        col = jnp.broadcast_to(
            lax.broadcasted_iota(jnp.int32, (1, n), 1), (PC, n))
        cand = jnp.where(sqd <= radius2, col, n)
        cols = []
        for _ in range(nsample):
            v = jnp.min(cand, axis=1, keepdims=True)
            cols.append(v)
            cand = jnp.where(cand == v, n, cand)
        idx = jnp.concatenate(cols, axis=1)
        first = idx[:, 0:1]
        first = jnp.where(first == n, 0, first)
        idx = jnp.where(idx == n, first, idx)
        o_ref[0] = idx + pl.program_id(0) * n

    return _pcall(
        body,
        grid=(B, P // PC),
        in_specs=[
            pl.BlockSpec((1, 8, n), lambda b, c: (b, 0, 0)),
            pl.BlockSpec((1, PC, 8), lambda b, c: (b, c, 0)),
        ],
        out_specs=pl.BlockSpec((1, PC, nsample), lambda b, c: (b, c, 0)),
        out_shape=jax.ShapeDtypeStruct((B, P, nsample), jnp.int32),
    )(xT, nx8)


def _three_nn(u_xyz, k_xyz):
    B, nu, _ = u_xyz.shape
    nk = k_xyz.shape[1]
    PC = min(256, nu)
    kT = jnp.zeros((B, 8, nk), jnp.float32).at[:, :3, :].set(
        jnp.transpose(k_xyz, (0, 2, 1)))
    u8 = jnp.zeros((B, nu, 8), jnp.float32).at[:, :, :3].set(u_xyz)

    def body(u_ref, k_ref, go_ref, do_ref):
        kx = k_ref[0, 0:1, :]
        ky = k_ref[0, 1:2, :]
        kz = k_ref[0, 2:3, :]
        u = u_ref[0]
        ux = u[:, 0:1]
        uy = u[:, 1:2]
        uz = u[:, 2:3]
        d = (ux - kx) ** 2 + (uy - ky) ** 2 + (uz - kz) ** 2
        col = jnp.broadcast_to(
            lax.broadcasted_iota(jnp.int32, (1, nk), 1), (PC, nk))
        idxs, ds = [], []
        for _ in range(3):
            v = jnp.min(d, axis=1, keepdims=True)
            j = jnp.min(jnp.where(d == v, col, nk), axis=1, keepdims=True)
            idxs.append(j)
            ds.append(v)
            d = jnp.where(col == j, jnp.inf, d)
        dist = jnp.concatenate(ds, axis=1)
        zi = jnp.zeros((PC, 5), jnp.int32)
        zf = jnp.zeros((PC, 5), jnp.float32)
        go_ref[0] = jnp.concatenate(idxs + [zi], axis=1) + pl.program_id(0) * nk
        do_ref[0] = jnp.concatenate([dist, zf], axis=1)

    return _pcall(
        body,
        grid=(B, nu // PC),
        in_specs=[
            pl.BlockSpec((1, PC, 8), lambda b, c: (b, c, 0)),
            pl.BlockSpec((1, 8, nk), lambda b, c: (b, 0, 0)),
        ],
        out_specs=[
            pl.BlockSpec((1, PC, 8), lambda b, c: (b, c, 0)),
            pl.BlockSpec((1, PC, 8), lambda b, c: (b, c, 0)),
        ],
        out_shape=[
            jax.ShapeDtypeStruct((B, nu, 8), jnp.int32),
            jax.ShapeDtypeStruct((B, nu, 8), jnp.float32),
        ],
    )(u8, kT)


def _mm_group(xg, nxp, wT, S):
    B, M, C = xg.shape
    P = M // S
    O = wT.shape[1]
    PC = min(64, P)
    RC = PC * S

    def body(x_ref, nx_ref, w_ref, y_ref):
        x = x_ref[0]
        nx = nx_ref[0]
        nxb = jnp.broadcast_to(nx[:, None, :], (PC, S, C)).reshape(RC, C)
        y_ref[0] = jnp.dot(x - nxb, w_ref[...],
                           preferred_element_type=jnp.float32)

    return _pcall(
        body,
        grid=(B, P // PC),
        in_specs=[
            pl.BlockSpec((1, RC, C), lambda b, c: (b, c, 0)),
            pl.BlockSpec((1, PC, C), lambda b, c: (b, c, 0)),
            pl.BlockSpec((C, O), lambda b, c: (0, 0)),
        ],
        out_specs=pl.BlockSpec((1, RC, O), lambda b, c: (b, c, 0)),
        out_shape=jax.ShapeDtypeStruct((B, M, O), jnp.float32),
    )(xg, nxp, wT)


def _mm(x, wT):
    B, M, C = x.shape
    O = wT.shape[1]
    RC = min(2048, M)

    def body(x_ref, w_ref, y_ref):
        y_ref[0] = jnp.dot(x_ref[0], w_ref[...],
                           preferred_element_type=jnp.float32)

    return _pcall(
        body,
        grid=(B, M // RC),
        in_specs=[
            pl.BlockSpec((1, RC, C), lambda b, c: (b, c, 0)),
            pl.BlockSpec((C, O), lambda b, c: (0, 0)),
        ],
        out_specs=pl.BlockSpec((1, RC, O), lambda b, c: (b, c, 0)),
        out_shape=jax.ShapeDtypeStruct((B, M, O), jnp.float32),
    )(x, wT)


def _mm_interp(g3, w8, skip, wT):
    B, M3, Og = g3.shape
    nu = M3 // 3
    Cs = skip.shape[2]
    O = wT.shape[1]
    PC = min(512, nu)

    def body(g_ref, w_ref, sk_ref, wt_ref, y_ref):
        g = g_ref[0].reshape(PC, 3, Og)
        w = w_ref[0]
        interp = (g[:, 0] * w[:, 0:1] + g[:, 1] * w[:, 1:2]
                  + g[:, 2] * w[:, 2:3])
        x = jnp.concatenate([interp, sk_ref[0]], axis=1)
        y_ref[0] = jnp.dot(x, wt_ref[...], preferred_element_type=jnp.float32)

    return _pcall(
        body,
        grid=(B, nu // PC),
        in_specs=[
            pl.BlockSpec((1, PC * 3, Og), lambda b, c: (b, c, 0)),
            pl.BlockSpec((1, PC, 8), lambda b, c: (b, c, 0)),
            pl.BlockSpec((1, PC, Cs), lambda b, c: (b, c, 0)),
            pl.BlockSpec((Og + Cs, O), lambda b, c: (0, 0)),
        ],
        out_specs=pl.BlockSpec((1, PC, O), lambda b, c: (b, c, 0)),
        out_shape=jax.ShapeDtypeStruct((B, nu, O), jnp.float32),
    )(g3, w8, skip, wT)


def _max_pool(x, S):
    B, M, O = x.shape
    P = M // S
    PC = min(64, P)

    def body(x_ref, o_ref):
        o_ref[0] = jnp.max(x_ref[0].reshape(PC, S, O), axis=1)

    return _pcall(
        body,
        grid=(B, P // PC),
        in_specs=[pl.BlockSpec((1, PC * S, O), lambda b, c: (b, c, 0))],
        out_specs=pl.BlockSpec((1, PC, O), lambda b, c: (b, c, 0)),
        out_shape=jax.ShapeDtypeStruct((B, P, O), jnp.float32),
    )(x)


def _mm_bias(x, wT, b):
    B, M, C = x.shape
    O = wT.shape[1]
    br = jnp.broadcast_to(b[None, :], (8, O))

    def body(x_ref, w_ref, b_ref, o_ref):
        o_ref[0] = jnp.dot(x_ref[0], w_ref[...],
                           preferred_element_type=jnp.float32) + b_ref[0:1, :]

    return _pcall(
        body,
        grid=(B, 1),
        in_specs=[
            pl.BlockSpec((1, M, C), lambda bb, c: (bb, 0, 0)),
            pl.BlockSpec((C, O), lambda bb, c: (0, 0)),
            pl.BlockSpec((8, O), lambda bb, c: (0, 0)),
        ],
        out_specs=pl.BlockSpec((1, M, O), lambda bb, c: (bb, 0, 0)),
        out_shape=jax.ShapeDtypeStruct((B, M, O), jnp.float32),
    )(x, wT, br)


def _bn_relu_2d(y, x_cm, W, g, be, P, S):
    B, M, O = y.shape
    ys = jnp.einsum('oc,bcns->bons', W, x_cm)
    m = jnp.mean(ys, axis=(0, 2, 3))
    v = jnp.var(ys, axis=(0, 2, 3))
    h = ((y - m[None, None, :]) / jnp.sqrt(v + 1e-5)[None, None, :]
         * g[None, None, :] + be[None, None, :])
    return jax.nn.relu(h)


def _bn_relu_1d(y, x_cm, W, g, be):
    B, M, O = y.shape
    ys = jnp.einsum('oc,bcn->bon', W, x_cm)
    m = jnp.mean(ys, axis=(0, 2))
    v = jnp.var(ys, axis=(0, 2))
    h = ((y - m[None, None, :]) / jnp.sqrt(v + 1e-5)[None, None, :]
         * g[None, None, :] + be[None, None, :])
    return jax.nn.relu(h)


def _pad_last(x, D):
    c = x.shape[-1]
    if c == D:
        return x
    return jnp.concatenate(
        [x, jnp.zeros(x.shape[:-1] + (D - c,), x.dtype)], axis=-1)


def kernel(pointcloud, params, numpoints):
    B, N, _ = pointcloud.shape
    S = 32
    NPT = (2048, 512, 128)
    RAD = (0.3, 0.5, 0.7)
    DP = (128, 256, 384)

    l_xyz = [pointcloud[..., 0:3]]
    l_feat = [pointcloud[..., 3:6]]
    tbl = _pad_last(pointcloud, DP[0])

    for i in range(3):
        n = l_xyz[i].shape[1]
        P = NPT[i]
        Dp = DP[i]
        fold = jnp.asarray(numpoints[i], jnp.int32) - P
        tbl_flat = tbl.reshape(B * n, Dp)

        gfidx = _fps(l_xyz[i], P, fold)
        rows = _gather_rows(tbl_flat, gfidx.reshape(-1)).reshape(B, P, Dp)
        new_xyz = rows[..., 0:3]
        nxp = _pad_last(new_xyz, Dp)

        gidx = _ball_query(l_xyz[i], new_xyz, np.float32(RAD[i] ** 2), S)
        xg = _gather_rows(tbl_flat, gidx.reshape(-1)).reshape(B, P * S, Dp)

        layers = params['sa'][i]
        (W1, g1, b1), (W2, g2, b2), (W3, g3, b3) = layers
        w1T = _pad_last(W1, Dp).T
        C1 = W1.shape[1]
        x1_cm = jnp.transpose(
            (xg - jnp.repeat(nxp, S, axis=1)).reshape(B, P, S, Dp),
            (0, 3, 1, 2))[:, :C1]
        y = _mm_group(xg, nxp, w1T, S)
        h = _bn_relu_2d(y, x1_cm, W1, g1, b1, P, S)
        x_cm = jnp.transpose(h.reshape(B, P, S, -1), (0, 3, 1, 2))
        h = _bn_relu_2d(_mm(h, W2.T), x_cm, W2, g2, b2, P, S)
        x_cm = jnp.transpose(h.reshape(B, P, S, -1), (0, 3, 1, 2))
        h = _bn_relu_2d(_mm(h, W3.T), x_cm, W3, g3, b3, P, S)
        feat = _max_pool(h, S)

        l_xyz.append(new_xyz)
        l_feat.append(feat)
        if i < 2:
            tbl = _pad_last(jnp.concatenate([new_xyz, feat], axis=-1),
                            DP[i + 1])

    Wt, bt = params['trans']
    mid_feat = jnp.transpose(_mm_bias(l_feat[3], Wt.T, bt), (0, 2, 1))
    mid_xyz = l_xyz[3]

    for i in range(3):
        u = 2 - i
        k = 3 - i
        nu = l_xyz[u].shape[1]
        nk = l_xyz[k].shape[1]
        gidx3, d8 = _three_nn(l_xyz[u], l_xyz[k])
        dist = d8[..., 0:3]
        dr = 1.0 / (dist + 1e-8)
        w = dr / jnp.sum(dr, axis=2, keepdims=True)
        w8 = _pad_last(w, 8)
        kf = l_feat[k]
        Og = kf.shape[2]
        g3 = _gather_rows(kf.reshape(B * nk, Og),
                          gidx3[..., 0:3].reshape(-1)).reshape(B, nu * 3, Og)
        skip = l_feat[u]
        Cs = skip.shape[2]
        Csp = 128 if Cs < 128 else Cs
        layers = params['fp'][i]
        (W1, g1, b1), (W2, g2, b2) = layers
        wT = jnp.concatenate(
            [W1[:, :Og], _pad_last(W1[:, Og:], Csp)], axis=1).T
        interp = jnp.sum(g3.reshape(B, nu, 3, Og) * w[..., None], axis=2)
        x1_cm = jnp.concatenate([jnp.transpose(interp, (0, 2, 1)),
                                 jnp.transpose(skip, (0, 2, 1))], axis=1)
        y = _mm_interp(g3, w8, _pad_last(skip, Csp), wT)
        h = _bn_relu_1d(y, x1_cm, W1, g1, b1)
        x_cm = jnp.transpose(h, (0, 2, 1))
        h = _bn_relu_1d(_mm(h, W2.T), x_cm, W2, g2, b2)
        l_feat[u] = h

    feat0 = jnp.transpose(l_feat[0], (0, 2, 1))
    return (l_xyz[0], feat0, mid_xyz, mid_feat)
